# Initial kernel scaffold; baseline (speedup 1.0000x reference)
import functools

import jax
import jax.numpy as jnp
from jax import lax
from jax.experimental import pallas as pl
from jax.experimental.pallas import tpu as pltpu
from jax.experimental.pallas import tpu_sc as plsc

_N = 10000
_E = 320000
_D = 128
_H = 32
_NC = 2
_NS = 16
_NW = _NC * _NS
_EPW = _E // _NW
_CH = 80
_NCH = _EPW // _CH
_RPS = _N // _NS


def _sc_aggregate_body(g_hbm, row_hbm, col_hbm, zeros_hbm, out_hbm,
                       row_v, col_v, buf0, buf1, acc, sem0, sem1):
    c = lax.axis_index("c")
    s = lax.axis_index("s")
    wid = c * _NS + s

    pltpu.sync_copy(zeros_hbm, acc.at[pl.ds(s * _RPS, _RPS)])
    pltpu.sync_copy(row_hbm.at[wid], row_v)
    pltpu.sync_copy(col_hbm.at[wid], col_v)
    plsc.subcore_barrier()

    pltpu.async_copy(g_hbm.at[row_v.at[0]], buf0, sem0)

    def step(i, carry):
        j = 2 * i
        pltpu.make_async_copy(g_hbm.at[row_v.at[j]], buf0, sem0).wait()
        pltpu.async_copy(g_hbm.at[row_v.at[j + 1]], buf1, sem1)
        pltpu.sync_copy(buf0, acc.at[col_v.at[j]], add=True)
        pltpu.make_async_copy(g_hbm.at[row_v.at[j + 1]], buf1, sem1).wait()
        pltpu.async_copy(g_hbm.at[row_v.at[j + 2]], buf0, sem0)
        pltpu.sync_copy(buf1, acc.at[col_v.at[j + 1]], add=True)
        return carry

    lax.fori_loop(0, (_NCH - 1) // 2, step, 0)
    pltpu.make_async_copy(g_hbm.at[row_v.at[_NCH - 1]], buf0, sem0).wait()
    pltpu.sync_copy(buf0, acc.at[col_v.at[_NCH - 1]], add=True)

    plsc.subcore_barrier()
    pltpu.sync_copy(acc.at[pl.ds(s * _RPS, _RPS)],
                    out_hbm.at[c, pl.ds(s * _RPS, _RPS)])


_sc_aggregate = functools.partial(
    pl.kernel,
    out_type=jax.ShapeDtypeStruct((_NC, _N, _H), jnp.float32),
    mesh=plsc.VectorSubcoreMesh(core_axis_name="c", subcore_axis_name="s"),
    scratch_types=[
        pltpu.VMEM((_NCH, _CH), jnp.int32),
        pltpu.VMEM((_NCH, _CH), jnp.int32),
        pltpu.VMEM((_CH, _H), jnp.float32),
        pltpu.VMEM((_CH, _H), jnp.float32),
        pltpu.VMEM_SHARED((_N, _H), jnp.float32),
        pltpu.SemaphoreType.DMA,
        pltpu.SemaphoreType.DMA,
    ],
)(_sc_aggregate_body)


def _tc_prologue_body(x0, w1, rawdeg, g_out, dinv_out):
    deg = rawdeg[0] + rawdeg[1] + 1.0
    dinv = lax.rsqrt(deg)
    h = jnp.dot(x0[...], w1[...], preferred_element_type=jnp.float32)
    dinv_out[...] = dinv
    g_out[...] = dinv * h


_tc_prologue = pl.pallas_call(
    _tc_prologue_body,
    out_shape=[jax.ShapeDtypeStruct((_N, _H), jnp.float32),
               jax.ShapeDtypeStruct((_N, _H), jnp.float32)],
)


def _tc_layer_body(raw, g, dinv, b, w, g_out):
    x = jnp.tanh(dinv[...] * (raw[0] + raw[1] + g[...]) + b[...])
    h = jnp.dot(x, w[...], preferred_element_type=jnp.float32)
    g_out[...] = dinv[...] * h


_tc_layer = pl.pallas_call(
    _tc_layer_body,
    out_shape=jax.ShapeDtypeStruct((_N, _H), jnp.float32),
)


def _tc_epilogue_body(raw, g, dinv, b2, w3, b3, noise, y_out):
    x = jnp.tanh(dinv[...] * (raw[0] + raw[1] + g[...]) + b2[...])
    y = jnp.tanh(jnp.dot(x, w3[...], preferred_element_type=jnp.float32)
                 + b3[...])
    y_out[...] = y + noise[...]


_tc_epilogue = pl.pallas_call(
    _tc_epilogue_body,
    out_shape=jax.ShapeDtypeStruct((_N, 1), jnp.float32),
)


def kernel(node_fea, edges, W1, b1, W2, b2, W3, b3):
    row3 = edges[0].astype(jnp.int32).reshape(_NW, _NCH, _CH)
    col3 = edges[1].astype(jnp.int32).reshape(_NW, _NCH, _CH)
    zeros = jnp.zeros((_RPS, _H), jnp.float32)
    ones = jnp.ones((_N, _H), jnp.float32)

    rawdeg = _sc_aggregate(ones, row3, col3, zeros)
    g, dinv = _tc_prologue(node_fea, W1, rawdeg)

    bb1 = b1.reshape(1, _H)
    bb2 = b2.reshape(1, _H)
    raw = _sc_aggregate(g, row3, col3, zeros)
    for l in range(20):
        g = _tc_layer(raw, g, dinv, bb1 if l == 0 else bb2, W2)
        raw = _sc_aggregate(g, row3, col3, zeros)

    noise = jax.random.normal(jax.random.key(42), (_N, 1), jnp.float32)
    return _tc_epilogue(raw, g, dinv, bb2, W3, b3.reshape(1, 1), noise)


# speedup vs baseline: 2.1504x; 2.1504x over previous
import functools

import jax
import jax.numpy as jnp
from jax import lax
from jax.experimental import pallas as pl
from jax.experimental.pallas import tpu as pltpu
from jax.experimental.pallas import tpu_sc as plsc

_N = 10000
_E = 320000
_H = 32
_FG = 4
_FW = _H // _FG
_EC = 8
_EPC = _E // _EC
_CH = 80
_NCH = _EPC // _CH
_NS = 16


def _sc_aggregate_body(g4_hbm, row_hbm, col_hbm, out_hbm,
                       rowbuf, colbuf, gbuf, acc, sem):
    c = lax.axis_index("c")
    s = lax.axis_index("s")
    wid = s * 2 + c
    fg = wid % _FG
    ec = wid // _FG
    foff = pl.multiple_of(fg * _FW, _FW)

    iota16 = lax.iota(jnp.int32, 16)

    zeros16 = jnp.zeros((16,), jnp.float32)

    def zstep(i, carry):
        rows = i * 16 + iota16
        for f in range(_FW):
            plsc.store_scatter(acc, [rows, jnp.full((16,), f, jnp.int32)],
                               zeros16)
        return carry

    lax.fori_loop(0, _N // 16, zstep, 0)

    def step(ch, carry):
        pltpu.sync_copy(row_hbm.at[ec, ch], rowbuf)
        pltpu.sync_copy(col_hbm.at[ec, ch], colbuf)
        pltpu.async_copy(g4_hbm.at[fg].at[rowbuf], gbuf, sem).wait()
        for p in range(_CH // 16):
            dst = colbuf[pl.ds(p * 16, 16)]
            src_rows = p * 16 + iota16
            for f in range(_FW):
                fvec = jnp.full((16,), f, jnp.int32)
                vals = plsc.load_gather(gbuf, [src_rows, fvec])
                plsc.addupdate_scatter(acc, [dst, fvec], vals)
        return carry

    lax.fori_loop(0, _NCH, step, 0)

    pltpu.sync_copy(acc, out_hbm.at[ec, :, pl.ds(foff, _FW)])


_sc_aggregate = functools.partial(
    pl.kernel,
    out_type=jax.ShapeDtypeStruct((_EC, _N, _H), jnp.float32),
    mesh=plsc.VectorSubcoreMesh(core_axis_name="c", subcore_axis_name="s"),
    compiler_params=pltpu.CompilerParams(use_tc_tiling_on_sc=False,
                                         needs_layout_passes=False),
    scratch_types=[
        pltpu.VMEM((_CH,), jnp.int32),
        pltpu.VMEM((_CH,), jnp.int32),
        pltpu.VMEM((_CH, _FW), jnp.float32),
        pltpu.VMEM((_N, _FW), jnp.float32),
        pltpu.SemaphoreType.DMA,
    ],
)(_sc_aggregate_body)


def _tc_prologue_body(x0, w1, rawdeg, g_out, dinv_out):
    deg = jnp.sum(rawdeg[...], axis=0) + 1.0
    dinv = lax.rsqrt(deg)
    h = jnp.dot(x0[...], w1[...], preferred_element_type=jnp.float32)
    dinv_out[...] = dinv
    g_out[...] = dinv * h


_BR = 2000
_NB = _N // _BR

_tc_prologue = pl.pallas_call(
    _tc_prologue_body,
    grid=(_NB,),
    in_specs=[
        pl.BlockSpec((_BR, 128), lambda i: (i, 0)),
        pl.BlockSpec((128, _H), lambda i: (0, 0)),
        pl.BlockSpec((_EC, _BR, _H), lambda i: (0, i, 0)),
    ],
    out_specs=[pl.BlockSpec((_BR, _H), lambda i: (i, 0)),
               pl.BlockSpec((_BR, _H), lambda i: (i, 0))],
    out_shape=[jax.ShapeDtypeStruct((_N, _H), jnp.float32),
               jax.ShapeDtypeStruct((_N, _H), jnp.float32)],
)


def _tc_layer_body(raw, g, dinv, b, w, g_out):
    agg = jnp.sum(raw[...], axis=0)
    x = jnp.tanh(dinv[...] * (agg + g[...]) + b[...])
    h = jnp.dot(x, w[...], preferred_element_type=jnp.float32)
    g_out[...] = dinv[...] * h


_tc_layer = pl.pallas_call(
    _tc_layer_body,
    grid=(_NB,),
    in_specs=[
        pl.BlockSpec((_EC, _BR, _H), lambda i: (0, i, 0)),
        pl.BlockSpec((_BR, _H), lambda i: (i, 0)),
        pl.BlockSpec((_BR, _H), lambda i: (i, 0)),
        pl.BlockSpec((1, _H), lambda i: (0, 0)),
        pl.BlockSpec((_H, _H), lambda i: (0, 0)),
    ],
    out_specs=pl.BlockSpec((_BR, _H), lambda i: (i, 0)),
    out_shape=jax.ShapeDtypeStruct((_N, _H), jnp.float32),
)


def _tc_epilogue_body(raw, g, dinv, b2, w3, b3, noise, y_out):
    agg = jnp.sum(raw[...], axis=0)
    x = jnp.tanh(dinv[...] * (agg + g[...]) + b2[...])
    y = jnp.tanh(jnp.dot(x, w3[...], preferred_element_type=jnp.float32)
                 + b3[...])
    y_out[...] = y + noise[...]


_tc_epilogue = pl.pallas_call(
    _tc_epilogue_body,
    grid=(_NB,),
    in_specs=[
        pl.BlockSpec((_EC, _BR, _H), lambda i: (0, i, 0)),
        pl.BlockSpec((_BR, _H), lambda i: (i, 0)),
        pl.BlockSpec((_BR, _H), lambda i: (i, 0)),
        pl.BlockSpec((1, _H), lambda i: (0, 0)),
        pl.BlockSpec((_H, 1), lambda i: (0, 0)),
        pl.BlockSpec((1, 1), lambda i: (0, 0)),
        pl.BlockSpec((_BR, 1), lambda i: (i, 0)),
    ],
    out_specs=pl.BlockSpec((_BR, 1), lambda i: (i, 0)),
    out_shape=jax.ShapeDtypeStruct((_N, 1), jnp.float32),
)


def kernel(node_fea, edges, W1, b1, W2, b2, W3, b3):
    row3 = edges[0].astype(jnp.int32).reshape(_EC, _NCH, _CH)
    col3 = edges[1].astype(jnp.int32).reshape(_EC, _NCH, _CH)

    def agg(g):
        g4 = jnp.transpose(g.reshape(_N, _FG, _FW), (1, 0, 2))
        return _sc_aggregate(g4, row3, col3)

    rawdeg = agg(jnp.ones((_N, _H), jnp.float32))
    g, dinv = _tc_prologue(node_fea, W1, rawdeg)

    bb1 = b1.reshape(1, _H)
    bb2 = b2.reshape(1, _H)
    raw = agg(g)
    for l in range(20):
        g = _tc_layer(raw, g, dinv, bb1 if l == 0 else bb2, W2)
        raw = agg(g)

    noise = jax.random.normal(jax.random.key(42), (_N, 1), jnp.float32)
    return _tc_epilogue(raw, g, dinv, bb2, W3, b3.reshape(1, 1), noise)


# speedup vs baseline: 6.4081x; 2.9800x over previous
import functools

import jax
import jax.numpy as jnp
from jax import lax
from jax.experimental import pallas as pl
from jax.experimental.pallas import tpu as pltpu
from jax.experimental.pallas import tpu_sc as plsc

_N = 10000
_E = 320000
_H = 32
_FG = 4
_FW = _H // _FG
_EC = 8
_EPC = _E // _EC
_CH = 80
_NCH = _EPC // _CH
_NS = 16


_SUP = 10
_NSUP = _NCH // _SUP
_SE = _SUP * _CH


def _sc_aggregate_body(g4_hbm, row_hbm, col_hbm, out_hbm,
                       rowb0, rowb1, colb0, colb1, gb0, gb1, acc,
                       semi0, semi1, semg0, semg1):
    c = lax.axis_index("c")
    s = lax.axis_index("s")
    wid = s * 2 + c
    fg = wid % _FG
    ec = wid // _FG
    foff = pl.multiple_of(fg * _FW, _FW)

    iota16 = lax.iota(jnp.int32, 16)

    zeros16 = jnp.zeros((16,), jnp.float32)

    def zstep(i, carry):
        rows = i * 16 + iota16
        for f in range(_FW):
            plsc.store_scatter(acc, [rows, jnp.full((16,), f, jnp.int32)],
                               zeros16)
        return carry

    lax.fori_loop(0, _N // 16, zstep, 0)

    def fire_idx(i, rowb, colb, semi):
        pltpu.async_copy(row_hbm.at[ec, pl.ds(i * _SUP, _SUP)], rowb, semi)
        pltpu.async_copy(col_hbm.at[ec, pl.ds(i * _SUP, _SUP)], colb, semi)

    def wait_idx(i, rowb, colb, semi):
        pltpu.make_async_copy(row_hbm.at[ec, pl.ds(i * _SUP, _SUP)], rowb,
                              semi).wait()
        pltpu.make_async_copy(col_hbm.at[ec, pl.ds(i * _SUP, _SUP)], colb,
                              semi).wait()

    def fire_gathers(rowb, gb, semg):
        for ch in range(_SUP):
            pltpu.async_copy(g4_hbm.at[fg].at[rowb.at[ch]],
                             gb.at[pl.ds(ch * _CH, _CH)], semg)

    def drain_gathers(rowb, gb, semg):
        for ch in range(_SUP):
            pltpu.make_async_copy(g4_hbm.at[fg].at[rowb.at[ch]],
                                  gb.at[pl.ds(ch * _CH, _CH)], semg).wait()

    def compute(colb, gb):
        def cstep(ch, carry):
            for p in range(_CH // 16):
                dst = colb[ch, pl.ds(p * 16, 16)]
                rows = ch * _CH + p * 16 + iota16
                for f in range(_FW):
                    fvec = jnp.full((16,), f, jnp.int32)
                    vals = plsc.load_gather(gb, [rows, fvec])
                    plsc.addupdate_scatter(acc, [dst, fvec], vals)
            return carry
        lax.fori_loop(0, _SUP, cstep, 0)

    slots = ((rowb0, colb0, gb0, semi0, semg0),
             (rowb1, colb1, gb1, semi1, semg1))

    fire_idx(0, rowb0, colb0, semi0)
    wait_idx(0, rowb0, colb0, semi0)
    fire_gathers(rowb0, gb0, semg0)
    fire_idx(1, rowb1, colb1, semi1)

    def body_one(i, slot_a, slot_b):
        rowa, cola, gba, semia, semga = slot_a
        rowbb, colbb, gbb, semib, semgb = slot_b
        drain_gathers(rowa, gba, semga)
        compute(cola, gba)

        @pl.when(i + 2 < _NSUP)
        def _():
            fire_idx(i + 2, rowa, cola, semia)

        @pl.when(i + 1 < _NSUP)
        def _():
            wait_idx(i + 1, rowbb, colbb, semib)
            fire_gathers(rowbb, gbb, semgb)

    def step(m, carry):
        body_one(2 * m, slots[0], slots[1])
        body_one(2 * m + 1, slots[1], slots[0])
        return carry

    lax.fori_loop(0, _NSUP // 2, step, 0)

    pltpu.sync_copy(acc, out_hbm.at[ec, :, pl.ds(foff, _FW)])


_sc_aggregate = functools.partial(
    pl.kernel,
    out_type=jax.ShapeDtypeStruct((_EC, _N, _H), jnp.float32),
    mesh=plsc.VectorSubcoreMesh(core_axis_name="c", subcore_axis_name="s"),
    compiler_params=pltpu.CompilerParams(use_tc_tiling_on_sc=False,
                                         needs_layout_passes=False),
    scratch_types=[
        pltpu.VMEM((_SUP, _CH), jnp.int32),
        pltpu.VMEM((_SUP, _CH), jnp.int32),
        pltpu.VMEM((_SUP, _CH), jnp.int32),
        pltpu.VMEM((_SUP, _CH), jnp.int32),
        pltpu.VMEM((_SE, _FW), jnp.float32),
        pltpu.VMEM((_SE, _FW), jnp.float32),
        pltpu.VMEM((_N, _FW), jnp.float32),
        pltpu.SemaphoreType.DMA,
        pltpu.SemaphoreType.DMA,
        pltpu.SemaphoreType.DMA,
        pltpu.SemaphoreType.DMA,
    ],
)(_sc_aggregate_body)


def _tc_prologue_body(x0, w1, rawdeg, g_out, dinv_out):
    deg = jnp.sum(rawdeg[...], axis=0) + 1.0
    dinv = lax.rsqrt(deg)
    h = jnp.dot(x0[...], w1[...], preferred_element_type=jnp.float32)
    dinv_out[...] = dinv
    g_out[...] = dinv * h


_BR = 2000
_NB = _N // _BR

_tc_prologue = pl.pallas_call(
    _tc_prologue_body,
    grid=(_NB,),
    in_specs=[
        pl.BlockSpec((_BR, 128), lambda i: (i, 0)),
        pl.BlockSpec((128, _H), lambda i: (0, 0)),
        pl.BlockSpec((_EC, _BR, _H), lambda i: (0, i, 0)),
    ],
    out_specs=[pl.BlockSpec((_BR, _H), lambda i: (i, 0)),
               pl.BlockSpec((_BR, _H), lambda i: (i, 0))],
    out_shape=[jax.ShapeDtypeStruct((_N, _H), jnp.float32),
               jax.ShapeDtypeStruct((_N, _H), jnp.float32)],
)


def _tc_layer_body(raw, g, dinv, b, w, g_out):
    agg = jnp.sum(raw[...], axis=0)
    x = jnp.tanh(dinv[...] * (agg + g[...]) + b[...])
    h = jnp.dot(x, w[...], preferred_element_type=jnp.float32)
    g_out[...] = dinv[...] * h


_tc_layer = pl.pallas_call(
    _tc_layer_body,
    grid=(_NB,),
    in_specs=[
        pl.BlockSpec((_EC, _BR, _H), lambda i: (0, i, 0)),
        pl.BlockSpec((_BR, _H), lambda i: (i, 0)),
        pl.BlockSpec((_BR, _H), lambda i: (i, 0)),
        pl.BlockSpec((1, _H), lambda i: (0, 0)),
        pl.BlockSpec((_H, _H), lambda i: (0, 0)),
    ],
    out_specs=pl.BlockSpec((_BR, _H), lambda i: (i, 0)),
    out_shape=jax.ShapeDtypeStruct((_N, _H), jnp.float32),
)


def _tc_epilogue_body(raw, g, dinv, b2, w3, b3, noise, y_out):
    agg = jnp.sum(raw[...], axis=0)
    x = jnp.tanh(dinv[...] * (agg + g[...]) + b2[...])
    y = jnp.tanh(jnp.dot(x, w3[...], preferred_element_type=jnp.float32)
                 + b3[...])
    y_out[...] = y + noise[...]


_tc_epilogue = pl.pallas_call(
    _tc_epilogue_body,
    grid=(_NB,),
    in_specs=[
        pl.BlockSpec((_EC, _BR, _H), lambda i: (0, i, 0)),
        pl.BlockSpec((_BR, _H), lambda i: (i, 0)),
        pl.BlockSpec((_BR, _H), lambda i: (i, 0)),
        pl.BlockSpec((1, _H), lambda i: (0, 0)),
        pl.BlockSpec((_H, 1), lambda i: (0, 0)),
        pl.BlockSpec((1, 1), lambda i: (0, 0)),
        pl.BlockSpec((_BR, 1), lambda i: (i, 0)),
    ],
    out_specs=pl.BlockSpec((_BR, 1), lambda i: (i, 0)),
    out_shape=jax.ShapeDtypeStruct((_N, 1), jnp.float32),
)


def kernel(node_fea, edges, W1, b1, W2, b2, W3, b3):
    row3 = edges[0].astype(jnp.int32).reshape(_EC, _NCH, _CH)
    col3 = edges[1].astype(jnp.int32).reshape(_EC, _NCH, _CH)

    def agg(g):
        g4 = jnp.transpose(g.reshape(_N, _FG, _FW), (1, 0, 2))
        return _sc_aggregate(g4, row3, col3)

    rawdeg = agg(jnp.ones((_N, _H), jnp.float32))
    g, dinv = _tc_prologue(node_fea, W1, rawdeg)

    bb1 = b1.reshape(1, _H)
    bb2 = b2.reshape(1, _H)
    raw = agg(g)
    for l in range(20):
        g = _tc_layer(raw, g, dinv, bb1 if l == 0 else bb2, W2)
        raw = agg(g)

    noise = jax.random.normal(jax.random.key(42), (_N, 1), jnp.float32)
    return _tc_epilogue(raw, g, dinv, bb2, W3, b3.reshape(1, 1), noise)
